# baseline (speedup 1.0000x reference)
import functools

import jax
import jax.numpy as jnp
from jax import lax
from jax.experimental import pallas as pl
from jax.experimental.pallas import tpu as pltpu
from jax.experimental.pallas import tpu_sc as plsc

NQ = 4
K = 8192
D = 256
TOK = 16 * 576
T = 512
KT = 2048
_HI = jax.lax.Precision.HIGHEST


def _compute_c2(cb_ref, c2_ref):
    ones = jnp.ones((1, D), jnp.float32)
    for t in range(K // KT):
        c = cb_ref[t * KT:(t + 1) * KT, :]
        c2_ref[:, t * KT:(t + 1) * KT] = jax.lax.dot_general(
            ones, c * c, (((1,), (1,)), ((), ())), precision=_HI,
            preferred_element_type=jnp.float32)


def _argmin_tiles(res, cb_bf_ref, c2_ref):
    r2 = jnp.sum(res * res, axis=1, keepdims=True)
    rb = (res * jnp.float32(-2.0)).astype(jnp.bfloat16)
    best = None
    bidx = None
    for t in range(K // KT):
        cb = cb_bf_ref[t * KT:(t + 1) * KT, :]
        e2 = jax.lax.dot_general(
            rb, cb, (((1,), (1,)), ((), ())),
            preferred_element_type=jnp.float32)
        dist = (r2 + e2) + c2_ref[:, t * KT:(t + 1) * KT]
        m = jnp.min(dist, axis=1, keepdims=True)
        li = jax.lax.broadcasted_iota(jnp.int32, (T, KT), 1).astype(jnp.float32)
        cand = jnp.min(jnp.where(dist == m, li, jnp.float32(1e9)),
                       axis=1, keepdims=True)
        gidx = cand + jnp.float32(t * KT)
        if t == 0:
            best, bidx = m, gidx
        else:
            take = m < best
            best = jnp.where(take, m, best)
            bidx = jnp.where(take, gidx, bidx)
    return bidx.astype(jnp.int32)


def _dist_first_body(z_ref, cb_bf_ref, cb_ref, idx_ref, c2_ref):
    @pl.when(pl.program_id(0) == 0)
    def _():
        _compute_c2(cb_ref, c2_ref)

    idx_ref[...] = _argmin_tiles(z_ref[...], cb_bf_ref, c2_ref)


def _dist_body(r_ref, q_ref, cb_bf_ref, cb_ref, res_ref, idx_ref, c2_ref):
    @pl.when(pl.program_id(0) == 0)
    def _():
        _compute_c2(cb_ref, c2_ref)

    res = r_ref[...] - q_ref[...]
    res_ref[...] = res
    idx_ref[...] = _argmin_tiles(res, cb_bf_ref, c2_ref)


def _dist_first(zf, cb_bf, cbq):
    return pl.pallas_call(
        _dist_first_body,
        grid=(TOK // T,),
        in_specs=[
            pl.BlockSpec((T, D), lambda i: (i, 0)),
            pl.BlockSpec((K, D), lambda i: (0, 0)),
            pl.BlockSpec((K, D), lambda i: (0, 0)),
        ],
        out_specs=pl.BlockSpec((T, 1), lambda i: (i, 0)),
        out_shape=jax.ShapeDtypeStruct((TOK, 1), jnp.int32),
        scratch_shapes=[pltpu.VMEM((1, K), jnp.float32)],
    )(zf, cb_bf, cbq)


def _dist_next(rprev, qprev, cb_bf, cbq):
    return pl.pallas_call(
        _dist_body,
        grid=(TOK // T,),
        in_specs=[
            pl.BlockSpec((T, D), lambda i: (i, 0)),
            pl.BlockSpec((T, D), lambda i: (i, 0)),
            pl.BlockSpec((K, D), lambda i: (0, 0)),
            pl.BlockSpec((K, D), lambda i: (0, 0)),
        ],
        out_specs=[
            pl.BlockSpec((T, D), lambda i: (i, 0)),
            pl.BlockSpec((T, 1), lambda i: (i, 0)),
        ],
        out_shape=[
            jax.ShapeDtypeStruct((TOK, D), jnp.float32),
            jax.ShapeDtypeStruct((TOK, 1), jnp.int32),
        ],
        scratch_shapes=[pltpu.VMEM((1, K), jnp.float32)],
    )(rprev, qprev, cb_bf, cbq)


def _sc_gather(table, idx):
    info = plsc.get_sparse_core_info()
    nc, ns = info.num_cores, info.num_subcores
    nw = nc * ns
    b_per_w = TOK // nw
    nch = -(-b_per_w // 96)
    ch = b_per_w // nch
    assert ch * nch == b_per_w and ch % 8 == 0 and ch <= 128
    mesh = plsc.VectorSubcoreMesh(core_axis_name="c", subcore_axis_name="s")

    @functools.partial(
        pl.kernel, mesh=mesh,
        out_type=jax.ShapeDtypeStruct((TOK, D), jnp.float32),
        scratch_types=[
            pltpu.VMEM((nch, ch), jnp.int32),
            pltpu.VMEM((b_per_w, D), jnp.float32),
            pltpu.SemaphoreType.DMA,
        ],
    )
    def k(table_hbm, idx_hbm, out_hbm, idx_v, rows_v, sem):
        wid = lax.axis_index("s") * nc + lax.axis_index("c")
        base = wid * b_per_w
        copies = []
        for j in range(nch):
            pltpu.sync_copy(idx_hbm.at[pl.ds(base + j * ch, ch)], idx_v.at[j])
            copies.append(pltpu.async_copy(
                table_hbm.at[idx_v.at[j]], rows_v.at[pl.ds(j * ch, ch)], sem))
        for c in copies:
            c.wait()
        pltpu.sync_copy(rows_v, out_hbm.at[pl.ds(base, b_per_w)])

    return k(table, idx)


def _epilogue_body(z_ref, r_ref, q_ref, out_ref, loss_ref):
    i = pl.program_id(0)
    z = z_ref[...]
    res = r_ref[...] - q_ref[...]
    qsum = z - res
    out_ref[...] = z + (qsum - z)
    part = jnp.sum(res * res).reshape(1, 1)

    @pl.when(i == 0)
    def _():
        loss_ref[...] = jnp.zeros((1, 1), jnp.float32)

    loss_ref[...] += part


def _epilogue(zf, r3, q3):
    return pl.pallas_call(
        _epilogue_body,
        grid=(TOK // T,),
        in_specs=[
            pl.BlockSpec((T, D), lambda i: (i, 0)),
            pl.BlockSpec((T, D), lambda i: (i, 0)),
            pl.BlockSpec((T, D), lambda i: (i, 0)),
        ],
        out_specs=[
            pl.BlockSpec((T, D), lambda i: (i, 0)),
            pl.BlockSpec((1, 1), lambda i: (0, 0)),
        ],
        out_shape=[
            jax.ShapeDtypeStruct((TOK, D), jnp.float32),
            jax.ShapeDtypeStruct((1, 1), jnp.float32),
        ],
    )(zf, r3, q3)


def kernel(z, codebooks):
    B, N = z.shape[0], z.shape[1]
    zf = z.reshape(TOK, D)
    cb_bf = codebooks.astype(jnp.bfloat16)

    idxs = []
    rprev = zf
    qprev = None
    for q in range(NQ):
        if q == 0:
            idxq = _dist_first(zf, cb_bf[0], codebooks[0])
        else:
            rprev, idxq = _dist_next(rprev, qprev, cb_bf[q], codebooks[q])
        idxs.append(idxq)
        qprev = _sc_gather(codebooks[q], idxq.reshape(TOK))

    out, losssum = _epilogue(zf, rprev, qprev)
    indices = jnp.concatenate(idxs, axis=1).reshape(B, N, NQ)
    m = losssum[0, 0] / (TOK * D)
    loss = 0.25 * m + m
    return out.reshape(B, N, D), indices, loss


# speedup vs baseline: 1.1812x; 1.1812x over previous
import functools

import jax
import jax.numpy as jnp
from jax import lax
from jax.experimental import pallas as pl
from jax.experimental.pallas import tpu as pltpu
from jax.experimental.pallas import tpu_sc as plsc

NQ = 4
K = 8192
D = 256
TOK = 16 * 576
T = 512
KT = 2048
_HI = jax.lax.Precision.HIGHEST


def _compute_c2(cb_ref, c2_ref):
    ones = jnp.ones((1, D), jnp.float32)
    for t in range(K // KT):
        c = cb_ref[t * KT:(t + 1) * KT, :]
        c2_ref[:, t * KT:(t + 1) * KT] = jax.lax.dot_general(
            ones, c * c, (((1,), (1,)), ((), ())), precision=_HI,
            preferred_element_type=jnp.float32)


def _argmin_tiles(res, cb_bf_ref, c2_ref):
    r2 = jnp.sum(res * res, axis=1, keepdims=True)
    rb = (res * jnp.float32(-2.0)).astype(jnp.bfloat16)
    run_min = jnp.full((T, 128), jnp.inf, jnp.float32)
    run_col = jnp.zeros((T, 128), jnp.float32)
    ncol = KT // 128
    for t in range(K // KT):
        cb = cb_bf_ref[t * KT:(t + 1) * KT, :]
        e2 = jax.lax.dot_general(
            rb, cb, (((1,), (1,)), ((), ())),
            preferred_element_type=jnp.float32)
        for c in range(ncol):
            lo = c * 128
            d_c = (r2 + e2[:, lo:lo + 128]) + c2_ref[:, t * KT + lo:t * KT + lo + 128]
            mask = d_c < run_min
            run_min = jnp.minimum(run_min, d_c)
            run_col = jnp.where(mask, jnp.float32(t * ncol + c), run_col)
    m = jnp.min(run_min, axis=1, keepdims=True)
    lane = jax.lax.broadcasted_iota(jnp.int32, (T, 128), 1).astype(jnp.float32)
    kc = run_col * jnp.float32(128.0) + lane
    cand = jnp.min(jnp.where(run_min == m, kc, jnp.float32(1e9)),
                   axis=1, keepdims=True)
    return cand.astype(jnp.int32)


def _dist_first_body(z_ref, cb_bf_ref, cb_ref, idx_ref, c2_ref):
    @pl.when(pl.program_id(0) == 0)
    def _():
        _compute_c2(cb_ref, c2_ref)

    idx_ref[...] = _argmin_tiles(z_ref[...], cb_bf_ref, c2_ref)


def _dist_body(r_ref, q_ref, cb_bf_ref, cb_ref, res_ref, idx_ref, c2_ref):
    @pl.when(pl.program_id(0) == 0)
    def _():
        _compute_c2(cb_ref, c2_ref)

    res = r_ref[...] - q_ref[...]
    res_ref[...] = res
    idx_ref[...] = _argmin_tiles(res, cb_bf_ref, c2_ref)


def _dist_first(zf, cb_bf, cbq):
    return pl.pallas_call(
        _dist_first_body,
        grid=(TOK // T,),
        in_specs=[
            pl.BlockSpec((T, D), lambda i: (i, 0)),
            pl.BlockSpec((K, D), lambda i: (0, 0)),
            pl.BlockSpec((K, D), lambda i: (0, 0)),
        ],
        out_specs=pl.BlockSpec((T, 1), lambda i: (i, 0)),
        out_shape=jax.ShapeDtypeStruct((TOK, 1), jnp.int32),
        scratch_shapes=[pltpu.VMEM((1, K), jnp.float32)],
    )(zf, cb_bf, cbq)


def _dist_next(rprev, qprev, cb_bf, cbq):
    return pl.pallas_call(
        _dist_body,
        grid=(TOK // T,),
        in_specs=[
            pl.BlockSpec((T, D), lambda i: (i, 0)),
            pl.BlockSpec((T, D), lambda i: (i, 0)),
            pl.BlockSpec((K, D), lambda i: (0, 0)),
            pl.BlockSpec((K, D), lambda i: (0, 0)),
        ],
        out_specs=[
            pl.BlockSpec((T, D), lambda i: (i, 0)),
            pl.BlockSpec((T, 1), lambda i: (i, 0)),
        ],
        out_shape=[
            jax.ShapeDtypeStruct((TOK, D), jnp.float32),
            jax.ShapeDtypeStruct((TOK, 1), jnp.int32),
        ],
        scratch_shapes=[pltpu.VMEM((1, K), jnp.float32)],
    )(rprev, qprev, cb_bf, cbq)


def _sc_gather(table, idx):
    info = plsc.get_sparse_core_info()
    nc, ns = info.num_cores, info.num_subcores
    nw = nc * ns
    b_per_w = TOK // nw
    nch = -(-b_per_w // 96)
    ch = b_per_w // nch
    assert ch * nch == b_per_w and ch % 8 == 0 and ch <= 128
    mesh = plsc.VectorSubcoreMesh(core_axis_name="c", subcore_axis_name="s")

    @functools.partial(
        pl.kernel, mesh=mesh,
        out_type=jax.ShapeDtypeStruct((TOK, D), jnp.float32),
        scratch_types=[
            pltpu.VMEM((nch, ch), jnp.int32),
            pltpu.VMEM((b_per_w, D), jnp.float32),
            pltpu.SemaphoreType.DMA,
        ],
    )
    def k(table_hbm, idx_hbm, out_hbm, idx_v, rows_v, sem):
        wid = lax.axis_index("s") * nc + lax.axis_index("c")
        base = wid * b_per_w
        copies = []
        for j in range(nch):
            pltpu.sync_copy(idx_hbm.at[pl.ds(base + j * ch, ch)], idx_v.at[j])
            copies.append(pltpu.async_copy(
                table_hbm.at[idx_v.at[j]], rows_v.at[pl.ds(j * ch, ch)], sem))
        for c in copies:
            c.wait()
        pltpu.sync_copy(rows_v, out_hbm.at[pl.ds(base, b_per_w)])

    return k(table, idx)


def _epilogue_body(z_ref, r_ref, q_ref, out_ref, loss_ref):
    i = pl.program_id(0)
    z = z_ref[...]
    res = r_ref[...] - q_ref[...]
    qsum = z - res
    out_ref[...] = z + (qsum - z)
    part = jnp.sum(res * res).reshape(1, 1)

    @pl.when(i == 0)
    def _():
        loss_ref[...] = jnp.zeros((1, 1), jnp.float32)

    loss_ref[...] += part


def _epilogue(zf, r3, q3):
    return pl.pallas_call(
        _epilogue_body,
        grid=(TOK // T,),
        in_specs=[
            pl.BlockSpec((T, D), lambda i: (i, 0)),
            pl.BlockSpec((T, D), lambda i: (i, 0)),
            pl.BlockSpec((T, D), lambda i: (i, 0)),
        ],
        out_specs=[
            pl.BlockSpec((T, D), lambda i: (i, 0)),
            pl.BlockSpec((1, 1), lambda i: (0, 0)),
        ],
        out_shape=[
            jax.ShapeDtypeStruct((TOK, D), jnp.float32),
            jax.ShapeDtypeStruct((1, 1), jnp.float32),
        ],
    )(zf, r3, q3)


def kernel(z, codebooks):
    B, N = z.shape[0], z.shape[1]
    zf = z.reshape(TOK, D)
    cb_bf = codebooks.astype(jnp.bfloat16)

    idxs = []
    rprev = zf
    qprev = None
    for q in range(NQ):
        if q == 0:
            idxq = _dist_first(zf, cb_bf[0], codebooks[0])
        else:
            rprev, idxq = _dist_next(rprev, qprev, cb_bf[q], codebooks[q])
        idxs.append(idxq)
        qprev = _sc_gather(codebooks[q], idxq.reshape(TOK))

    out, losssum = _epilogue(zf, rprev, qprev)
    indices = jnp.concatenate(idxs, axis=1).reshape(B, N, NQ)
    m = losssum[0, 0] / (TOK * D)
    loss = 0.25 * m + m
    return out.reshape(B, N, D), indices, loss


# speedup vs baseline: 1.2458x; 1.0547x over previous
import functools

import jax
import jax.numpy as jnp
from jax import lax
from jax.experimental import pallas as pl
from jax.experimental.pallas import tpu as pltpu
from jax.experimental.pallas import tpu_sc as plsc

NQ = 4
K = 8192
D = 256
TOK = 16 * 576
T = 1024
KT = 2048
_HI = jax.lax.Precision.HIGHEST


def _compute_c2(cb_ref, c2_ref):
    ones = jnp.ones((1, D), jnp.float32)
    for t in range(K // KT):
        c = cb_ref[t * KT:(t + 1) * KT, :]
        c2_ref[:, t * KT:(t + 1) * KT] = jax.lax.dot_general(
            ones, c * c, (((1,), (1,)), ((), ())), precision=_HI,
            preferred_element_type=jnp.float32)


def _argmin_tiles(res, cb_bf_ref, c2_ref):
    r2 = jnp.sum(res * res, axis=1, keepdims=True)
    rb = (res * jnp.float32(-2.0)).astype(jnp.bfloat16)
    run_min = jnp.full((T, 128), jnp.inf, jnp.float32)
    run_col = jnp.zeros((T, 128), jnp.float32)
    ncol = KT // 128
    for t in range(K // KT):
        cb = cb_bf_ref[t * KT:(t + 1) * KT, :]
        e2 = jax.lax.dot_general(
            rb, cb, (((1,), (1,)), ((), ())),
            preferred_element_type=jnp.float32)
        for c in range(ncol):
            lo = c * 128
            d_c = (r2 + e2[:, lo:lo + 128]) + c2_ref[:, t * KT + lo:t * KT + lo + 128]
            mask = d_c < run_min
            run_min = jnp.minimum(run_min, d_c)
            run_col = jnp.where(mask, jnp.float32(t * ncol + c), run_col)
    m = jnp.min(run_min, axis=1, keepdims=True)
    lane = jax.lax.broadcasted_iota(jnp.int32, (T, 128), 1).astype(jnp.float32)
    kc = run_col * jnp.float32(128.0) + lane
    cand = jnp.min(jnp.where(run_min == m, kc, jnp.float32(1e9)),
                   axis=1, keepdims=True)
    return cand.astype(jnp.int32)


def _dist_first_body(z_ref, cb_bf_ref, cb_ref, idx_ref, c2_ref):
    @pl.when(pl.program_id(0) == 0)
    def _():
        _compute_c2(cb_ref, c2_ref)

    idx_ref[...] = _argmin_tiles(z_ref[...], cb_bf_ref, c2_ref)


def _dist_body(r_ref, q_ref, cb_bf_ref, cb_ref, res_ref, idx_ref, c2_ref):
    @pl.when(pl.program_id(0) == 0)
    def _():
        _compute_c2(cb_ref, c2_ref)

    res = r_ref[...] - q_ref[...]
    res_ref[...] = res
    idx_ref[...] = _argmin_tiles(res, cb_bf_ref, c2_ref)


def _dist_first(zf, cb_bf, cbq):
    return pl.pallas_call(
        _dist_first_body,
        grid=(TOK // T,),
        in_specs=[
            pl.BlockSpec((T, D), lambda i: (i, 0)),
            pl.BlockSpec((K, D), lambda i: (0, 0)),
            pl.BlockSpec((K, D), lambda i: (0, 0)),
        ],
        out_specs=pl.BlockSpec((T, 1), lambda i: (i, 0)),
        out_shape=jax.ShapeDtypeStruct((TOK, 1), jnp.int32),
        scratch_shapes=[pltpu.VMEM((1, K), jnp.float32)],
    )(zf, cb_bf, cbq)


def _dist_next(rprev, qprev, cb_bf, cbq):
    return pl.pallas_call(
        _dist_body,
        grid=(TOK // T,),
        in_specs=[
            pl.BlockSpec((T, D), lambda i: (i, 0)),
            pl.BlockSpec((T, D), lambda i: (i, 0)),
            pl.BlockSpec((K, D), lambda i: (0, 0)),
            pl.BlockSpec((K, D), lambda i: (0, 0)),
        ],
        out_specs=[
            pl.BlockSpec((T, D), lambda i: (i, 0)),
            pl.BlockSpec((T, 1), lambda i: (i, 0)),
        ],
        out_shape=[
            jax.ShapeDtypeStruct((TOK, D), jnp.float32),
            jax.ShapeDtypeStruct((TOK, 1), jnp.int32),
        ],
        scratch_shapes=[pltpu.VMEM((1, K), jnp.float32)],
    )(rprev, qprev, cb_bf, cbq)


def _sc_gather(table, idx):
    info = plsc.get_sparse_core_info()
    nc, ns = info.num_cores, info.num_subcores
    nw = nc * ns
    b_per_w = TOK // nw
    nch = -(-b_per_w // 96)
    ch = b_per_w // nch
    assert ch * nch == b_per_w and ch % 8 == 0 and ch <= 128
    mesh = plsc.VectorSubcoreMesh(core_axis_name="c", subcore_axis_name="s")

    @functools.partial(
        pl.kernel, mesh=mesh,
        out_type=jax.ShapeDtypeStruct((TOK, D), jnp.float32),
        scratch_types=[
            pltpu.VMEM((nch, ch), jnp.int32),
            pltpu.VMEM((b_per_w, D), jnp.float32),
            pltpu.SemaphoreType.DMA,
        ],
    )
    def k(table_hbm, idx_hbm, out_hbm, idx_v, rows_v, sem):
        wid = lax.axis_index("s") * nc + lax.axis_index("c")
        base = wid * b_per_w
        copies = []
        for j in range(nch):
            pltpu.sync_copy(idx_hbm.at[pl.ds(base + j * ch, ch)], idx_v.at[j])
            copies.append(pltpu.async_copy(
                table_hbm.at[idx_v.at[j]], rows_v.at[pl.ds(j * ch, ch)], sem))
        for c in copies:
            c.wait()
        pltpu.sync_copy(rows_v, out_hbm.at[pl.ds(base, b_per_w)])

    return k(table, idx)


def _epilogue_body(z_ref, r_ref, q_ref, out_ref, loss_ref):
    i = pl.program_id(0)
    z = z_ref[...]
    res = r_ref[...] - q_ref[...]
    qsum = z - res
    out_ref[...] = z + (qsum - z)
    part = jnp.sum(res * res).reshape(1, 1)

    @pl.when(i == 0)
    def _():
        loss_ref[...] = jnp.zeros((1, 1), jnp.float32)

    loss_ref[...] += part


def _epilogue(zf, r3, q3):
    return pl.pallas_call(
        _epilogue_body,
        grid=(TOK // T,),
        in_specs=[
            pl.BlockSpec((T, D), lambda i: (i, 0)),
            pl.BlockSpec((T, D), lambda i: (i, 0)),
            pl.BlockSpec((T, D), lambda i: (i, 0)),
        ],
        out_specs=[
            pl.BlockSpec((T, D), lambda i: (i, 0)),
            pl.BlockSpec((1, 1), lambda i: (0, 0)),
        ],
        out_shape=[
            jax.ShapeDtypeStruct((TOK, D), jnp.float32),
            jax.ShapeDtypeStruct((1, 1), jnp.float32),
        ],
    )(zf, r3, q3)


def kernel(z, codebooks):
    B, N = z.shape[0], z.shape[1]
    zf = z.reshape(TOK, D)
    cb_bf = codebooks.astype(jnp.bfloat16)

    idxs = []
    rprev = zf
    qprev = None
    for q in range(NQ):
        if q == 0:
            idxq = _dist_first(zf, cb_bf[0], codebooks[0])
        else:
            rprev, idxq = _dist_next(rprev, qprev, cb_bf[q], codebooks[q])
        idxs.append(idxq)
        qprev = _sc_gather(codebooks[q], idxq.reshape(TOK))

    out, losssum = _epilogue(zf, rprev, qprev)
    indices = jnp.concatenate(idxs, axis=1).reshape(B, N, NQ)
    m = losssum[0, 0] / (TOK * D)
    loss = 0.25 * m + m
    return out.reshape(B, N, D), indices, loss
